# baseline (speedup 1.0000x reference)
import jax
import jax.numpy as jnp
from jax.experimental import pallas as pl

B = 32
N_J = 50
N_M = 20
N = N_J * N_M
D = 64
H = 64

_NEG_INF = float("-inf")


def _body(adj_ref, x_ref, gp_ref, cand_ref, maskf_ref, aidx_ref, oa_ref,
          dur_ref, mm_ref, mch_ref,
          gW01_ref, gb01_ref, gW02_ref, gb02_ref,
          gW11_ref, gb11_ref, gW12_ref, gb12_ref,
          aW1_ref, ab1_ref, aW2_ref, ab2_ref, aW3_ref, ab3_ref,
          cW1_ref, cb1_ref, cW2_ref, cb2_ref,
          ent_ref, v_ref, loga_ref, anode_ref, afeat_ref, mma_ref,
          hpool_ref):
    f32 = jnp.float32
    bf16 = jnp.bfloat16
    adj = adj_ref[0].astype(bf16)
    xg = x_ref[0].astype(bf16)

    p0 = jnp.dot(adj, xg, preferred_element_type=f32)
    t0 = jnp.maximum(jnp.dot(p0.astype(bf16), gW01_ref[...].astype(bf16),
                             preferred_element_type=f32)
                     + gb01_ref[...], 0.0)
    h1 = jnp.maximum(jnp.dot(t0.astype(bf16), gW02_ref[...].astype(bf16),
                             preferred_element_type=f32)
                     + gb02_ref[...], 0.0)
    p1 = jnp.dot(adj, h1.astype(bf16), preferred_element_type=f32)
    t1 = jnp.maximum(jnp.dot(p1.astype(bf16), gW11_ref[...].astype(bf16),
                             preferred_element_type=f32)
                     + gb11_ref[...], 0.0)
    h2 = jnp.maximum(jnp.dot(t1.astype(bf16), gW12_ref[...].astype(bf16),
                             preferred_element_type=f32)
                     + gb12_ref[...], 0.0)

    gp = gp_ref[0]
    h_pooled = jnp.dot(gp, h2, preferred_element_type=f32)

    cand = cand_ref[0]
    iota_n = jax.lax.broadcasted_iota(jnp.int32, (N_J, N), 1)
    onehot = (iota_n == cand).astype(f32)
    cand_feat = jnp.dot(onehot, h2, preferred_element_type=f32)

    mch = mch_ref[0]
    a1 = jnp.tanh(
        jnp.dot(cand_feat, aW1_ref[0:H, :], preferred_element_type=f32)
        + jnp.dot(h_pooled, aW1_ref[H:2 * H, :], preferred_element_type=f32)
        + jnp.dot(mch, aW1_ref[2 * H:3 * H, :], preferred_element_type=f32)
        + ab1_ref[...])
    a2 = jnp.tanh(jnp.dot(a1, aW2_ref[...], preferred_element_type=f32)
                  + ab2_ref[...])
    scores = (jnp.dot(a2, aW3_ref[...], preferred_element_type=f32)
              + ab3_ref[...]) * 10.0
    scores = jnp.where(maskf_ref[0] > 0.5, _NEG_INF, scores)

    m = jnp.max(scores, axis=0, keepdims=True)
    e = jnp.exp(scores - m)
    z = jnp.sum(e, axis=0, keepdims=True)
    log_pi = scores - m - jnp.log(z)
    pi = jnp.exp(log_pi)
    ent_ref[0] = -jnp.sum(pi * log_pi, axis=0, keepdims=True)

    aidx = aidx_ref[0]
    iota_j = jax.lax.broadcasted_iota(jnp.int32, (N_J, 1), 0)
    oh_a = (iota_j == aidx).astype(f32)
    loga_ref[0] = jnp.sum(log_pi * oh_a, axis=0, keepdims=True)

    oa = oa_ref[0]
    iota_row = jax.lax.broadcasted_iota(jnp.int32, (1, N), 1)
    oh_o = (iota_row == oa).astype(f32)
    afeat_ref[0] = jnp.dot(oh_o, h2, preferred_element_type=f32)
    anode_ref[0] = jnp.dot(oh_o, dur_ref[0], preferred_element_type=f32)
    mmf = mm_ref[0].astype(f32)
    mma_ref[0] = jnp.dot(oh_o, mmf, preferred_element_type=f32) > 0.5

    c1 = jnp.tanh(jnp.dot(h_pooled, cW1_ref[...], preferred_element_type=f32)
                  + cb1_ref[...])
    v_ref[0] = (jnp.dot(c1, cW2_ref[...], preferred_element_type=f32)
                + cb2_ref[...])
    hpool_ref[0] = h_pooled


def _b3(shape):
    return pl.BlockSpec((1,) + shape, lambda b: (b, 0, 0))


def _w(shape):
    return pl.BlockSpec(shape, lambda b: (0,) * len(shape))


def kernel(x, graph_pool, padded_nei, adj, candidate, mask, mask_mch, dur,
           a_index, old_action, mch_pool,
           gW01, gb01, gW02, gb02, gW11, gb11, gW12, gb12,
           aW1, ab1, aW2, ab2, aW3, ab3, cW1, cb1, cW2, cb2):
    f32 = jnp.float32
    gp3 = graph_pool.reshape(B, 1, N)
    cand3 = candidate.astype(jnp.int32).reshape(B, N_J, 1)
    maskf3 = mask.astype(f32).reshape(B, N_J, 1)
    a3 = a_index.astype(jnp.int32).reshape(B, 1, 1)
    oa3 = old_action.astype(jnp.int32).reshape(B, 1, 1)
    mm_i8 = mask_mch.astype(jnp.int8)
    mch3 = mch_pool.reshape(B, 1, H)
    biases2 = [b.reshape(1, -1) for b in
               (gb01, gb02, gb11, gb12, ab1, ab2, ab3, cb1, cb2)]
    gb01r, gb02r, gb11r, gb12r, ab1r, ab2r, ab3r, cb1r, cb2r = biases2

    out_shapes = (
        jax.ShapeDtypeStruct((B, 1, 1), f32),
        jax.ShapeDtypeStruct((B, 1, 1), f32),
        jax.ShapeDtypeStruct((B, 1, 1), f32),
        jax.ShapeDtypeStruct((B, 1, N_M), f32),
        jax.ShapeDtypeStruct((B, 1, H), f32),
        jax.ShapeDtypeStruct((B, 1, N_M), jnp.bool_),
        jax.ShapeDtypeStruct((B, 1, H), f32),
    )
    in_specs = [
        _b3((N, N)),
        _b3((N, D)),
        _b3((1, N)),
        _b3((N_J, 1)),
        _b3((N_J, 1)),
        _b3((1, 1)),
        _b3((1, 1)),
        _b3((N, N_M)),
        _b3((N, N_M)),
        _b3((1, H)),
        _w((D, H)), _w((1, H)), _w((H, H)), _w((1, H)),
        _w((H, H)), _w((1, H)), _w((H, H)), _w((1, H)),
        _w((3 * H, H)), _w((1, H)), _w((H, H)), _w((1, H)),
        _w((H, 1)), _w((1, 1)), _w((H, H)), _w((1, H)),
        _w((H, 1)), _w((1, 1)),
    ]
    out_specs = (
        _b3((1, 1)), _b3((1, 1)), _b3((1, 1)), _b3((1, N_M)),
        _b3((1, H)), _b3((1, N_M)), _b3((1, H)),
    )
    ent, v, loga, anode, afeat, mma, hpool = pl.pallas_call(
        _body,
        grid=(B,),
        in_specs=in_specs,
        out_specs=out_specs,
        out_shape=out_shapes,
    )(adj, x, gp3, cand3, maskf3, a3, oa3, dur, mm_i8, mch3,
      gW01, gb01r, gW02, gb02r, gW11, gb11r, gW12, gb12r,
      aW1, ab1r, aW2, ab2r, aW3, ab3r, cW1, cb1r, cW2, cb2r)

    return (ent.reshape(B), v.reshape(B, 1), loga.reshape(B),
            anode.reshape(B, N_M), afeat.reshape(B, H),
            mma, hpool.reshape(B, H))


# speedup vs baseline: 1.0096x; 1.0096x over previous
import jax
import jax.numpy as jnp
from jax.experimental import pallas as pl

B = 32
N_J = 50
N_M = 20
N = N_J * N_M
D = 64
H = 64
G = 2

_NEG_INF = float("-inf")


def _per_graph(g, adj_ref, x_ref, gp_ref, cand_ref, maskf_ref, aidx_ref,
               oa_ref, dur_ref, mm_ref, mch_ref,
               gW01_ref, gb01_ref, gW02_ref, gb02_ref,
               gW11_ref, gb11_ref, gW12_ref, gb12_ref,
               aW1_ref, ab1_ref, aW2_ref, ab2_ref, aW3_ref, ab3_ref,
               cW1_ref, cb1_ref, cW2_ref, cb2_ref,
               ent_ref, v_ref, loga_ref, anode_ref, afeat_ref, mma_ref,
               hpool_ref):
    f32 = jnp.float32
    bf16 = jnp.bfloat16
    adj = adj_ref[g].astype(bf16)
    xg = x_ref[g].astype(bf16)

    p0 = jnp.dot(adj, xg, preferred_element_type=f32)
    t0 = jnp.maximum(jnp.dot(p0.astype(bf16), gW01_ref[...].astype(bf16),
                             preferred_element_type=f32)
                     + gb01_ref[...], 0.0)
    h1 = jnp.maximum(jnp.dot(t0.astype(bf16), gW02_ref[...].astype(bf16),
                             preferred_element_type=f32)
                     + gb02_ref[...], 0.0)
    p1 = jnp.dot(adj, h1.astype(bf16), preferred_element_type=f32)
    t1 = jnp.maximum(jnp.dot(p1.astype(bf16), gW11_ref[...].astype(bf16),
                             preferred_element_type=f32)
                     + gb11_ref[...], 0.0)
    h2 = jnp.maximum(jnp.dot(t1.astype(bf16), gW12_ref[...].astype(bf16),
                             preferred_element_type=f32)
                     + gb12_ref[...], 0.0)

    gp = gp_ref[g]
    h_pooled = jnp.dot(gp, h2, preferred_element_type=f32)

    cand = cand_ref[g]
    iota_n = jax.lax.broadcasted_iota(jnp.int32, (N_J, N), 1)
    onehot = (iota_n == cand).astype(f32)
    cand_feat = jnp.dot(onehot, h2, preferred_element_type=f32)

    mch = mch_ref[g]
    a1 = jnp.tanh(
        jnp.dot(cand_feat, aW1_ref[0:H, :], preferred_element_type=f32)
        + jnp.dot(h_pooled, aW1_ref[H:2 * H, :], preferred_element_type=f32)
        + jnp.dot(mch, aW1_ref[2 * H:3 * H, :], preferred_element_type=f32)
        + ab1_ref[...])
    a2 = jnp.tanh(jnp.dot(a1, aW2_ref[...], preferred_element_type=f32)
                  + ab2_ref[...])
    scores = (jnp.dot(a2, aW3_ref[...], preferred_element_type=f32)
              + ab3_ref[...]) * 10.0
    scores = jnp.where(maskf_ref[g] > 0.5, _NEG_INF, scores)

    m = jnp.max(scores, axis=0, keepdims=True)
    e = jnp.exp(scores - m)
    z = jnp.sum(e, axis=0, keepdims=True)
    log_pi = scores - m - jnp.log(z)
    pi = jnp.exp(log_pi)
    ent_ref[g] = -jnp.sum(pi * log_pi, axis=0, keepdims=True)

    aidx = aidx_ref[g]
    iota_j = jax.lax.broadcasted_iota(jnp.int32, (N_J, 1), 0)
    oh_a = (iota_j == aidx).astype(f32)
    loga_ref[g] = jnp.sum(log_pi * oh_a, axis=0, keepdims=True)

    oa = oa_ref[g]
    iota_row = jax.lax.broadcasted_iota(jnp.int32, (1, N), 1)
    oh_o = (iota_row == oa).astype(f32)
    afeat_ref[g] = jnp.dot(oh_o, h2, preferred_element_type=f32)
    anode_ref[g] = jnp.dot(oh_o, dur_ref[g], preferred_element_type=f32)
    mmf = mm_ref[g].astype(f32)
    mma_ref[g] = jnp.dot(oh_o, mmf, preferred_element_type=f32) > 0.5

    c1 = jnp.tanh(jnp.dot(h_pooled, cW1_ref[...], preferred_element_type=f32)
                  + cb1_ref[...])
    v_ref[g] = (jnp.dot(c1, cW2_ref[...], preferred_element_type=f32)
                + cb2_ref[...])
    hpool_ref[g] = h_pooled


def _body(*refs):
    for g in range(G):
        _per_graph(g, *refs)


def _b3(shape):
    return pl.BlockSpec((G,) + shape, lambda b: (b, 0, 0))


def _w(shape):
    return pl.BlockSpec(shape, lambda b: (0,) * len(shape))


def kernel(x, graph_pool, padded_nei, adj, candidate, mask, mask_mch, dur,
           a_index, old_action, mch_pool,
           gW01, gb01, gW02, gb02, gW11, gb11, gW12, gb12,
           aW1, ab1, aW2, ab2, aW3, ab3, cW1, cb1, cW2, cb2):
    f32 = jnp.float32
    gp3 = graph_pool.reshape(B, 1, N)
    cand3 = candidate.astype(jnp.int32).reshape(B, N_J, 1)
    maskf3 = mask.astype(f32).reshape(B, N_J, 1)
    a3 = a_index.astype(jnp.int32).reshape(B, 1, 1)
    oa3 = old_action.astype(jnp.int32).reshape(B, 1, 1)
    mm_i8 = mask_mch.astype(jnp.int8)
    mch3 = mch_pool.reshape(B, 1, H)
    biases2 = [b.reshape(1, -1) for b in
               (gb01, gb02, gb11, gb12, ab1, ab2, ab3, cb1, cb2)]
    gb01r, gb02r, gb11r, gb12r, ab1r, ab2r, ab3r, cb1r, cb2r = biases2

    out_shapes = (
        jax.ShapeDtypeStruct((B, 1, 1), f32),
        jax.ShapeDtypeStruct((B, 1, 1), f32),
        jax.ShapeDtypeStruct((B, 1, 1), f32),
        jax.ShapeDtypeStruct((B, 1, N_M), f32),
        jax.ShapeDtypeStruct((B, 1, H), f32),
        jax.ShapeDtypeStruct((B, 1, N_M), jnp.bool_),
        jax.ShapeDtypeStruct((B, 1, H), f32),
    )
    in_specs = [
        _b3((N, N)),
        _b3((N, D)),
        _b3((1, N)),
        _b3((N_J, 1)),
        _b3((N_J, 1)),
        _b3((1, 1)),
        _b3((1, 1)),
        _b3((N, N_M)),
        _b3((N, N_M)),
        _b3((1, H)),
        _w((D, H)), _w((1, H)), _w((H, H)), _w((1, H)),
        _w((H, H)), _w((1, H)), _w((H, H)), _w((1, H)),
        _w((3 * H, H)), _w((1, H)), _w((H, H)), _w((1, H)),
        _w((H, 1)), _w((1, 1)), _w((H, H)), _w((1, H)),
        _w((H, 1)), _w((1, 1)),
    ]
    out_specs = (
        _b3((1, 1)), _b3((1, 1)), _b3((1, 1)), _b3((1, N_M)),
        _b3((1, H)), _b3((1, N_M)), _b3((1, H)),
    )
    ent, v, loga, anode, afeat, mma, hpool = pl.pallas_call(
        _body,
        grid=(B // G,),
        in_specs=in_specs,
        out_specs=out_specs,
        out_shape=out_shapes,
    )(adj, x, gp3, cand3, maskf3, a3, oa3, dur, mm_i8, mch3,
      gW01, gb01r, gW02, gb02r, gW11, gb11r, gW12, gb12r,
      aW1, ab1r, aW2, ab2r, aW3, ab3r, cW1, cb1r, cW2, cb2r)

    return (ent.reshape(B), v.reshape(B, 1), loga.reshape(B),
            anode.reshape(B, N_M), afeat.reshape(B, H),
            mma, hpool.reshape(B, H))


# speedup vs baseline: 1.2999x; 1.2875x over previous
import jax
import jax.numpy as jnp
from jax.experimental import pallas as pl

B = 32
N_J = 50
N_M = 20
N = N_J * N_M
D = 64
H = 64
G = 2

_NEG_INF = float("-inf")

_WD01, _WD02, _WD11, _WD12 = 0, 128, 256, 384
_A1C, _A1H, _A1M, _AW2, _CW1 = 0, 64, 128, 192, 256
_B01, _B02, _B11, _B12 = 320, 321, 322, 323
_AB1, _AB2, _CB1, _AW3, _CW2, _MISC = 324, 325, 326, 327, 328, 329


def _body(adj_ref, x_ref, gm_ref, idx_ref, maskf_ref, dur_ref, mm_ref,
          wd_ref, ws_ref,
          ent_ref, v_ref, loga_ref, anode_ref, afeat_ref, mma_ref,
          hpool_ref):
    f32 = jnp.float32
    bf16 = jnp.bfloat16

    adjs = [adj_ref[g].astype(bf16) for g in range(G)]
    p0s = [jnp.dot(adjs[g], x_ref[g].astype(bf16),
                   preferred_element_type=f32) for g in range(G)]
    p0 = jnp.concatenate(p0s, axis=1)

    t0 = jnp.maximum(jnp.dot(p0.astype(bf16), wd_ref[_WD01:_WD01 + 128, :],
                             preferred_element_type=f32)
                     + ws_ref[_B01:_B01 + 1, :], 0.0)
    h1 = jnp.maximum(jnp.dot(t0.astype(bf16), wd_ref[_WD02:_WD02 + 128, :],
                             preferred_element_type=f32)
                     + ws_ref[_B02:_B02 + 1, :], 0.0)

    p1s = [jnp.dot(adjs[g], h1[:, g * H:(g + 1) * H].astype(bf16),
                   preferred_element_type=f32) for g in range(G)]
    p1 = jnp.concatenate(p1s, axis=1)
    t1 = jnp.maximum(jnp.dot(p1.astype(bf16), wd_ref[_WD11:_WD11 + 128, :],
                             preferred_element_type=f32)
                     + ws_ref[_B11:_B11 + 1, :], 0.0)
    h2 = jnp.maximum(jnp.dot(t1.astype(bf16), wd_ref[_WD12:_WD12 + 128, :],
                             preferred_element_type=f32)
                     + ws_ref[_B12:_B12 + 1, :], 0.0)

    for g in range(G):
        h2g = h2[:, g * H:(g + 1) * H]
        gp = gm_ref[g][:, 0:N]
        mch = gm_ref[g][:, N:N + H]
        h_pooled = jnp.dot(gp, h2g, preferred_element_type=f32)

        cand = idx_ref[g][0:N_J, :]
        iota_n = jax.lax.broadcasted_iota(jnp.int32, (N_J, N), 1)
        onehot = (iota_n == cand).astype(f32)
        cand_feat = jnp.dot(onehot, h2g, preferred_element_type=f32)

        a1 = jnp.tanh(
            jnp.dot(cand_feat, ws_ref[_A1C:_A1C + H, 0:H],
                    preferred_element_type=f32)
            + jnp.dot(h_pooled, ws_ref[_A1H:_A1H + H, 0:H],
                      preferred_element_type=f32)
            + jnp.dot(mch, ws_ref[_A1M:_A1M + H, 0:H],
                      preferred_element_type=f32)
            + ws_ref[_AB1:_AB1 + 1, 0:H])
        a2 = jnp.tanh(jnp.dot(a1, ws_ref[_AW2:_AW2 + H, 0:H],
                              preferred_element_type=f32)
                      + ws_ref[_AB2:_AB2 + 1, 0:H])
        ab3 = ws_ref[_MISC:_MISC + 1, 0:1]
        scores = (jnp.sum(a2 * ws_ref[_AW3:_AW3 + 1, 0:H], axis=1,
                          keepdims=True) + ab3) * 10.0
        scores = jnp.where(maskf_ref[g] > 0.5, _NEG_INF, scores)

        m = jnp.max(scores, axis=0, keepdims=True)
        e = jnp.exp(scores - m)
        z = jnp.sum(e, axis=0, keepdims=True)
        log_pi = scores - m - jnp.log(z)
        pi = jnp.exp(log_pi)
        ent_ref[g] = -jnp.sum(pi * log_pi, axis=0, keepdims=True)

        aidx = idx_ref[g][N_J:N_J + 1, :]
        iota_j = jax.lax.broadcasted_iota(jnp.int32, (N_J, 1), 0)
        oh_a = (iota_j == aidx).astype(f32)
        loga_ref[g] = jnp.sum(log_pi * oh_a, axis=0, keepdims=True)

        oa = idx_ref[g][N_J + 1:N_J + 2, :]
        iota_row = jax.lax.broadcasted_iota(jnp.int32, (1, N), 1)
        oh_o = (iota_row == oa).astype(f32)
        afeat_ref[g] = jnp.dot(oh_o, h2g, preferred_element_type=f32)
        anode_ref[g] = jnp.dot(oh_o, dur_ref[g], preferred_element_type=f32)
        mmf = mm_ref[g].astype(f32)
        mma_ref[g] = jnp.dot(oh_o, mmf, preferred_element_type=f32) > 0.5

        c1 = jnp.tanh(jnp.dot(h_pooled, ws_ref[_CW1:_CW1 + H, 0:H],
                              preferred_element_type=f32)
                      + ws_ref[_CB1:_CB1 + 1, 0:H])
        cb2 = ws_ref[_MISC:_MISC + 1, 1:2]
        v_ref[g] = (jnp.sum(c1 * ws_ref[_CW2:_CW2 + 1, 0:H], axis=1,
                            keepdims=True) + cb2)
        hpool_ref[g] = h_pooled


def _b3(shape):
    return pl.BlockSpec((G,) + shape, lambda b: (b, 0, 0))


def _w(shape):
    return pl.BlockSpec(shape, lambda b: (0,) * len(shape))


def kernel(x, graph_pool, padded_nei, adj, candidate, mask, mask_mch, dur,
           a_index, old_action, mch_pool,
           gW01, gb01, gW02, gb02, gW11, gb11, gW12, gb12,
           aW1, ab1, aW2, ab2, aW3, ab3, cW1, cb1, cW2, cb2):
    f32 = jnp.float32
    bf16 = jnp.bfloat16
    i32 = jnp.int32

    zw = jnp.zeros((D, H), f32)

    def bd(w):
        return jnp.concatenate(
            [jnp.concatenate([w, zw], 1), jnp.concatenate([zw, w], 1)], 0)

    wd = jnp.concatenate(
        [bd(gW01), bd(gW02), bd(gW11), bd(gW12)], 0).astype(bf16)

    def pad128(w):
        return jnp.concatenate([w, jnp.zeros_like(w)], 1)

    def row128(vec64):
        return jnp.concatenate([vec64, jnp.zeros((H,), f32)])[None, :]

    misc = jnp.concatenate([ab3, cb2, jnp.zeros((126,), f32)])[None, :]
    ws = jnp.concatenate(
        [pad128(aW1), pad128(aW2), pad128(cW1),
         jnp.concatenate([gb01, gb01])[None, :],
         jnp.concatenate([gb02, gb02])[None, :],
         jnp.concatenate([gb11, gb11])[None, :],
         jnp.concatenate([gb12, gb12])[None, :],
         row128(ab1), row128(ab2), row128(cb1),
         row128(aW3[:, 0]), row128(cW2[:, 0]), misc], 0)

    gm = jnp.concatenate([graph_pool, mch_pool], axis=1)[:, None, :]
    idx_all = jnp.concatenate(
        [candidate.astype(i32), a_index.astype(i32)[:, None],
         old_action.astype(i32)[:, None]], axis=1)[:, :, None]
    maskf3 = mask.astype(f32)[:, :, None]

    out_shapes = (
        jax.ShapeDtypeStruct((B, 1, 1), f32),
        jax.ShapeDtypeStruct((B, 1, 1), f32),
        jax.ShapeDtypeStruct((B, 1, 1), f32),
        jax.ShapeDtypeStruct((B, 1, N_M), f32),
        jax.ShapeDtypeStruct((B, 1, H), f32),
        jax.ShapeDtypeStruct((B, 1, N_M), jnp.bool_),
        jax.ShapeDtypeStruct((B, 1, H), f32),
    )
    in_specs = [
        _b3((N, N)),
        _b3((N, D)),
        _b3((1, N + H)),
        _b3((N_J + 2, 1)),
        _b3((N_J, 1)),
        _b3((N, N_M)),
        _b3((N, N_M)),
        _w((512, 128)),
        _w((330, 128)),
    ]
    out_specs = (
        _b3((1, 1)), _b3((1, 1)), _b3((1, 1)), _b3((1, N_M)),
        _b3((1, H)), _b3((1, N_M)), _b3((1, H)),
    )
    ent, v, loga, anode, afeat, mma, hpool = pl.pallas_call(
        _body,
        grid=(B // G,),
        in_specs=in_specs,
        out_specs=out_specs,
        out_shape=out_shapes,
    )(adj, x, gm, idx_all, maskf3, dur, mask_mch, wd, ws)

    return (ent.reshape(B), v.reshape(B, 1), loga.reshape(B),
            anode.reshape(B, N_M), afeat.reshape(B, H),
            mma, hpool.reshape(B, H))
